# baseline (speedup 1.0000x reference)
import functools

import numpy as np
import jax
import jax.numpy as jnp
from jax import lax
from jax.experimental import pallas as pl
from jax.experimental.pallas import tpu as pltpu
from jax.experimental.pallas import tpu_sc as plsc

GRID = 16
LOG2_HASH = 19
FEAT = 32
PRIMES = (1, 2654435761, 805459861)

NC = 2
NS = 16
L = 16
NW = NC * NS


def _cell_hashes() -> np.ndarray:
    ii = np.arange(GRID, dtype=np.int64)
    h = (
        (ii[:, None, None] * PRIMES[0])
        ^ (ii[None, :, None] * PRIMES[1])
        ^ (ii[None, None, :] * PRIMES[2])
    ) % (2 ** LOG2_HASH)
    return h.reshape(-1).astype(np.int32)


_HASHES = _cell_hashes()
_NCELL = GRID ** 3


def _gather_feats(table):
    b_per_w = _NCELL // NW
    mesh = plsc.VectorSubcoreMesh(core_axis_name="c", subcore_axis_name="s")
    idx = jnp.asarray(_HASHES)

    @functools.partial(
        pl.kernel,
        mesh=mesh,
        out_type=jax.ShapeDtypeStruct((_NCELL, FEAT), jnp.float32),
        scratch_types=[
            pltpu.VMEM((b_per_w,), jnp.int32),
            pltpu.VMEM((b_per_w, FEAT), jnp.float32),
            pltpu.SemaphoreType.DMA,
        ],
        compiler_params=pltpu.CompilerParams(use_tc_tiling_on_sc=False),
    )
    def k(table_hbm, idx_hbm, out_hbm, idx_v, rows_v, sem):
        wid = lax.axis_index("s") * NC + lax.axis_index("c")
        base = wid * b_per_w
        pltpu.sync_copy(idx_hbm.at[pl.ds(base, b_per_w)], idx_v)
        pltpu.async_copy(table_hbm.at[idx_v], rows_v, sem).wait()
        pltpu.sync_copy(rows_v, out_hbm.at[pl.ds(base, b_per_w)])

    return k(table, idx)


def _mlp_lut(feats, W1, b1, W2, b2, W3, b3):
    W3p = jnp.zeros((64, 8), jnp.float32).at[:, :3].set(W3)
    b3p = jnp.zeros((1, 8), jnp.float32).at[:, :3].set(b3)

    def body(f, w1, b1r, w2, b2r, w3, b3r, o):
        h1 = jnp.maximum(
            jnp.dot(f[...], w1[...], preferred_element_type=jnp.float32)
            + b1r[...], 0.0
        )
        h2 = jnp.maximum(
            jnp.dot(h1, w2[...], preferred_element_type=jnp.float32)
            + b2r[...], 0.0
        )
        o[...] = jax.nn.sigmoid(
            jnp.dot(h2, w3[...], preferred_element_type=jnp.float32)
            + b3r[...]
        )

    return pl.pallas_call(
        body,
        out_shape=jax.ShapeDtypeStruct((_NCELL, 8), jnp.float32),
    )(feats, W1, b1.reshape(1, 64), W2, b2.reshape(1, 64), W3p, b3p)


def _point_colors(xyz2d, lutr, lutg, lutb):
    n = xyz2d.shape[0] * xyz2d.shape[1] // 3
    pps = n // NC
    C2 = 65536
    n_chunks = pps // C2
    CT = C2 // NS
    W2 = 3 * C2
    WT = 3 * CT
    R2 = W2 // 128
    RT = WT // 128
    mesh = plsc.VectorSubcoreMesh(core_axis_name="c", subcore_axis_name="s")

    @functools.partial(
        pl.kernel,
        mesh=mesh,
        out_type=jax.ShapeDtypeStruct((3 * n,), jnp.float32),
        scratch_types=[
            pltpu.VMEM((_NCELL,), jnp.float32),
            pltpu.VMEM((_NCELL,), jnp.float32),
            pltpu.VMEM((_NCELL,), jnp.float32),
            pltpu.VMEM((RT, 128), jnp.float32),
            pltpu.VMEM((WT,), jnp.float32),
            pltpu.VMEM_SHARED((3 * _NCELL,), jnp.float32),
            pltpu.VMEM_SHARED((R2, 128), jnp.float32),
            pltpu.VMEM_SHARED((W2,), jnp.float32),
        ],
        compiler_params=pltpu.CompilerParams(needs_layout_passes=False),
    )
    def k(xyz_hbm, lr_hbm, lg_hbm, lb_hbm, out_hbm,
          lr_v, lg_v, lb_v, in_v, out_v, sh_lut, sh_in, sh_out):
        cid = lax.axis_index("c")
        sid = lax.axis_index("s")

        @pl.when(sid == 0)
        def _():
            pltpu.sync_copy(lr_hbm, sh_lut.at[pl.ds(0, _NCELL)])
            pltpu.sync_copy(lg_hbm, sh_lut.at[pl.ds(_NCELL, _NCELL)])
            pltpu.sync_copy(lb_hbm, sh_lut.at[pl.ds(2 * _NCELL, _NCELL)])
        plsc.subcore_barrier()
        pltpu.sync_copy(sh_lut.at[pl.ds(0, _NCELL)], lr_v)
        pltpu.sync_copy(sh_lut.at[pl.ds(_NCELL, _NCELL)], lg_v)
        pltpu.sync_copy(sh_lut.at[pl.ds(2 * _NCELL, _NCELL)], lb_v)

        pbase = cid * pps
        lane1 = lax.iota(jnp.int32, L)
        RP = C2 // 128
        RPT = CT // 128

        def drain(ch):
            p0 = pbase + ch * C2
            pltpu.sync_copy(sh_out.at[pl.ds(0, C2)], out_hbm.at[pl.ds(p0, C2)])
            pltpu.sync_copy(
                sh_out.at[pl.ds(C2, C2)], out_hbm.at[pl.ds(n + p0, C2)]
            )
            pltpu.sync_copy(
                sh_out.at[pl.ds(2 * C2, C2)], out_hbm.at[pl.ds(2 * n + p0, C2)]
            )

        for ch in range(n_chunks):
            prow = (pbase + ch * C2) // 128
            nrow = n // 128

            @pl.when(sid == 0)
            def _():
                for c in range(3):
                    pltpu.sync_copy(
                        xyz_hbm.at[
                            pl.ds(pl.multiple_of(c * nrow + prow, 8), RP), :
                        ],
                        sh_in.at[pl.ds(c * RP, RP), :],
                    )
            if ch > 0:
                @pl.when(sid == 1)
                def _():
                    drain(ch - 1)
            plsc.subcore_barrier()
            for c in range(3):
                pltpu.sync_copy(
                    sh_in.at[
                        pl.ds(pl.multiple_of(c * RP + sid * RPT, 8), RPT), :
                    ],
                    in_v.at[pl.ds(c * RPT, RPT), :],
                )

            @plsc.parallel_loop(
                np.int32(0), np.int32(CT // L), np.int32(1), unroll=8,
                carry=lane1,
            )
            def body(v, aj):
                row = aj >> 7
                col = aj & 127
                x = plsc.load_gather(in_v, [row, col])
                y = plsc.load_gather(in_v, [row + RPT, col])
                z = plsc.load_gather(in_v, [row + 2 * RPT, col])
                xi = (x * 16.0).astype(jnp.int32)
                yi = (y * 16.0).astype(jnp.int32)
                zi = (z * 16.0).astype(jnp.int32)
                cell = xi * 256 + yi * 16 + zi
                plsc.store_scatter(out_v, [aj], plsc.load_gather(lr_v, [cell]))
                plsc.store_scatter(
                    out_v, [aj + CT], plsc.load_gather(lg_v, [cell])
                )
                plsc.store_scatter(
                    out_v, [aj + 2 * CT], plsc.load_gather(lb_v, [cell])
                )
                return aj + L

            pltpu.sync_copy(
                out_v.at[pl.ds(0, CT)], sh_out.at[pl.ds(sid * CT, CT)]
            )
            pltpu.sync_copy(
                out_v.at[pl.ds(CT, CT)], sh_out.at[pl.ds(C2 + sid * CT, CT)]
            )
            pltpu.sync_copy(
                out_v.at[pl.ds(2 * CT, CT)],
                sh_out.at[pl.ds(2 * C2 + sid * CT, CT)],
            )
            plsc.subcore_barrier()

        @pl.when(sid == 1)
        def _():
            drain(n_chunks - 1)

    return k(xyz2d, lutr, lutg, lutb)


def _density_sigmoid(dg):
    flat = dg.reshape(16384, 128)

    def body(x, o):
        o[...] = jax.nn.sigmoid(x[...])

    out = pl.pallas_call(
        body,
        out_shape=jax.ShapeDtypeStruct((16384, 128), jnp.float32),
    )(flat)
    return out.reshape(128, 128, 128)


def kernel(xyz, tables, density_grid, W1, b1, W2, b2, W3, b3):
    f32 = jnp.float32
    color_dtype = jnp.result_type(
        xyz.dtype, tables.dtype, W1.dtype, b1.dtype, W2.dtype,
        b2.dtype, W3.dtype, b3.dtype,
    )
    density_dtype = density_grid.dtype
    xyz = xyz.astype(f32)
    density_grid = density_grid.astype(f32)
    W1, b1, W2, b2, W3, b3 = (
        a.astype(f32) for a in (W1, b1, W2, b2, W3, b3)
    )
    feats = _gather_feats(tables[0].astype(f32))
    lut = _mlp_lut(feats, W1, b1, W2, b2, W3, b3)
    colors_flat = _point_colors(
        xyz.T.reshape(-1, 128), lut[:, 0], lut[:, 1], lut[:, 2]
    )
    color = colors_flat.reshape(3, -1).astype(color_dtype).T
    density = _density_sigmoid(density_grid).astype(density_dtype)
    return (density, color)


# speedup vs baseline: 1.3841x; 1.3841x over previous
import functools

import numpy as np
import jax
import jax.numpy as jnp
from jax import lax
from jax.experimental import pallas as pl
from jax.experimental.pallas import tpu as pltpu
from jax.experimental.pallas import tpu_sc as plsc

GRID = 16
LOG2_HASH = 19
FEAT = 32
PRIMES = (1, 2654435761, 805459861)

NC = 2
NS = 16
L = 16
NW = NC * NS


def _cell_hashes() -> np.ndarray:
    ii = np.arange(GRID, dtype=np.int64)
    h = (
        (ii[:, None, None] * PRIMES[0])
        ^ (ii[None, :, None] * PRIMES[1])
        ^ (ii[None, None, :] * PRIMES[2])
    ) % (2 ** LOG2_HASH)
    return h.reshape(-1).astype(np.int32)


_HASHES = _cell_hashes()
_NCELL = GRID ** 3


def _gather_feats_t(table_t):
    b_per_w = _NCELL // NW
    mesh = plsc.VectorSubcoreMesh(core_axis_name="c", subcore_axis_name="s")
    idx = jnp.asarray(_HASHES)

    @functools.partial(
        pl.kernel,
        mesh=mesh,
        out_type=jax.ShapeDtypeStruct((NW, FEAT, b_per_w), jnp.float32),
        scratch_types=[
            pltpu.VMEM((b_per_w,), jnp.int32),
            pltpu.VMEM((FEAT, b_per_w), jnp.float32),
            pltpu.SemaphoreType.DMA,
        ],
        compiler_params=pltpu.CompilerParams(
            use_tc_tiling_on_sc=False, needs_layout_passes=False
        ),
    )
    def k(table_hbm, idx_hbm, out_hbm, idx_v, rows_v, sem):
        wid = lax.axis_index("s") * NC + lax.axis_index("c")
        base = wid * b_per_w
        pltpu.sync_copy(idx_hbm.at[pl.ds(base, b_per_w)], idx_v)
        for c in range(FEAT):
            pltpu.async_copy(
                table_hbm.at[np.int32(c)].at[idx_v],
                rows_v.at[np.int32(c)], sem,
            ).wait()
        pltpu.sync_copy(rows_v, out_hbm.at[wid])

    return k(table_t, idx)


def _mlp_lut_t(feats_t, W1, b1, W2, b2, W3, b3):
    W3p = jnp.zeros((64, 8), jnp.float32).at[:, :3].set(W3)
    b3p = jnp.zeros((8, 1), jnp.float32).at[:3, 0].set(b3)

    def body(f, w1t, b1r, w2t, b2r, w3t, b3r, o):
        h1 = jnp.maximum(
            jnp.dot(w1t[...], f[...], preferred_element_type=jnp.float32)
            + b1r[...], 0.0
        )
        h2 = jnp.maximum(
            jnp.dot(w2t[...], h1, preferred_element_type=jnp.float32)
            + b2r[...], 0.0
        )
        o[...] = jax.nn.sigmoid(
            jnp.dot(w3t[...], h2, preferred_element_type=jnp.float32)
            + b3r[...]
        )

    return pl.pallas_call(
        body,
        out_shape=jax.ShapeDtypeStruct((8, _NCELL), jnp.float32),
    )(
        feats_t, W1.T, b1.reshape(64, 1), W2.T, b2.reshape(64, 1),
        W3p.T, b3p,
    )


def _point_colors(xyz2d, lutr, lutg, lutb):
    n = xyz2d.shape[0] * xyz2d.shape[1] // 3
    pps = n // NC
    C2 = 65536
    n_chunks = pps // C2
    CT = C2 // NS
    W2 = 3 * C2
    WT = 3 * CT
    R2 = W2 // 128
    RT = WT // 128
    mesh = plsc.VectorSubcoreMesh(core_axis_name="c", subcore_axis_name="s")

    @functools.partial(
        pl.kernel,
        mesh=mesh,
        out_type=jax.ShapeDtypeStruct((3 * n,), jnp.float32),
        scratch_types=[
            pltpu.VMEM((_NCELL,), jnp.float32),
            pltpu.VMEM((_NCELL,), jnp.float32),
            pltpu.VMEM((_NCELL,), jnp.float32),
            pltpu.VMEM((RT, 128), jnp.float32),
            pltpu.VMEM((WT,), jnp.float32),
            pltpu.VMEM_SHARED((3 * _NCELL,), jnp.float32),
            pltpu.VMEM_SHARED((R2, 128), jnp.float32),
            pltpu.VMEM_SHARED((W2,), jnp.float32),
        ],
        compiler_params=pltpu.CompilerParams(needs_layout_passes=False),
    )
    def k(xyz_hbm, lr_hbm, lg_hbm, lb_hbm, out_hbm,
          lr_v, lg_v, lb_v, in_v, out_v, sh_lut, sh_in, sh_out):
        cid = lax.axis_index("c")
        sid = lax.axis_index("s")

        @pl.when(sid == 0)
        def _():
            pltpu.sync_copy(lr_hbm, sh_lut.at[pl.ds(0, _NCELL)])
            pltpu.sync_copy(lg_hbm, sh_lut.at[pl.ds(_NCELL, _NCELL)])
            pltpu.sync_copy(lb_hbm, sh_lut.at[pl.ds(2 * _NCELL, _NCELL)])
        plsc.subcore_barrier()
        pltpu.sync_copy(sh_lut.at[pl.ds(0, _NCELL)], lr_v)
        pltpu.sync_copy(sh_lut.at[pl.ds(_NCELL, _NCELL)], lg_v)
        pltpu.sync_copy(sh_lut.at[pl.ds(2 * _NCELL, _NCELL)], lb_v)

        pbase = cid * pps
        lane1 = lax.iota(jnp.int32, L)
        RP = C2 // 128
        RPT = CT // 128

        def drain(ch):
            p0 = pbase + ch * C2
            pltpu.sync_copy(sh_out.at[pl.ds(0, C2)], out_hbm.at[pl.ds(p0, C2)])
            pltpu.sync_copy(
                sh_out.at[pl.ds(C2, C2)], out_hbm.at[pl.ds(n + p0, C2)]
            )
            pltpu.sync_copy(
                sh_out.at[pl.ds(2 * C2, C2)], out_hbm.at[pl.ds(2 * n + p0, C2)]
            )

        for ch in range(n_chunks):
            prow = (pbase + ch * C2) // 128
            nrow = n // 128

            @pl.when(sid == 0)
            def _():
                for c in range(3):
                    pltpu.sync_copy(
                        xyz_hbm.at[
                            pl.ds(pl.multiple_of(c * nrow + prow, 8), RP), :
                        ],
                        sh_in.at[pl.ds(c * RP, RP), :],
                    )
            if ch > 0:
                @pl.when(sid == 1)
                def _():
                    drain(ch - 1)
            plsc.subcore_barrier()
            for c in range(3):
                pltpu.sync_copy(
                    sh_in.at[
                        pl.ds(pl.multiple_of(c * RP + sid * RPT, 8), RPT), :
                    ],
                    in_v.at[pl.ds(c * RPT, RPT), :],
                )

            @plsc.parallel_loop(
                np.int32(0), np.int32(CT // L), np.int32(1), unroll=8,
                carry=lane1,
            )
            def body(v, aj):
                row = aj >> 7
                col = aj & 127
                x = plsc.load_gather(in_v, [row, col])
                y = plsc.load_gather(in_v, [row + RPT, col])
                z = plsc.load_gather(in_v, [row + 2 * RPT, col])
                xi = (x * 16.0).astype(jnp.int32)
                yi = (y * 16.0).astype(jnp.int32)
                zi = (z * 16.0).astype(jnp.int32)
                cell = xi * 256 + yi * 16 + zi
                plsc.store_scatter(out_v, [aj], plsc.load_gather(lr_v, [cell]))
                plsc.store_scatter(
                    out_v, [aj + CT], plsc.load_gather(lg_v, [cell])
                )
                plsc.store_scatter(
                    out_v, [aj + 2 * CT], plsc.load_gather(lb_v, [cell])
                )
                return aj + L

            pltpu.sync_copy(
                out_v.at[pl.ds(0, CT)], sh_out.at[pl.ds(sid * CT, CT)]
            )
            pltpu.sync_copy(
                out_v.at[pl.ds(CT, CT)], sh_out.at[pl.ds(C2 + sid * CT, CT)]
            )
            pltpu.sync_copy(
                out_v.at[pl.ds(2 * CT, CT)],
                sh_out.at[pl.ds(2 * C2 + sid * CT, CT)],
            )
            plsc.subcore_barrier()

        @pl.when(sid == 1)
        def _():
            drain(n_chunks - 1)

    return k(xyz2d, lutr, lutg, lutb)


def _density_sigmoid(dg):
    flat = dg.reshape(16384, 128)

    def body(x, o):
        o[...] = jax.nn.sigmoid(x[...])

    out = pl.pallas_call(
        body,
        out_shape=jax.ShapeDtypeStruct((16384, 128), jnp.float32),
    )(flat)
    return out.reshape(128, 128, 128)


def kernel(xyz, tables, density_grid, W1, b1, W2, b2, W3, b3):
    f32 = jnp.float32
    color_dtype = jnp.result_type(
        xyz.dtype, tables.dtype, W1.dtype, b1.dtype, W2.dtype,
        b2.dtype, W3.dtype, b3.dtype,
    )
    density_dtype = density_grid.dtype
    xyz = xyz.astype(f32)
    density_grid = density_grid.astype(f32)
    W1, b1, W2, b2, W3, b3 = (
        a.astype(f32) for a in (W1, b1, W2, b2, W3, b3)
    )
    table_t = tables[0].astype(f32).T
    fw = _gather_feats_t(table_t)
    feats_t = fw.transpose(1, 0, 2).reshape(FEAT, _NCELL)
    lut = _mlp_lut_t(feats_t, W1, b1, W2, b2, W3, b3)
    colors_flat = _point_colors(
        xyz.T.reshape(-1, 128), lut[0], lut[1], lut[2]
    )
    color = colors_flat.reshape(3, -1).astype(color_dtype).T
    density = _density_sigmoid(density_grid).astype(density_dtype)
    return (density, color)
